# baseline (speedup 1.0000x reference)
import functools

import jax
import jax.numpy as jnp
from jax import lax
from jax.experimental import pallas as pl
from jax.experimental.pallas import tpu as pltpu
from jax.experimental.pallas import tpu_sc as plsc

VOCAB = 100000
EMB = 128
SEQ = 200
BATCH = 4096
NTOK = BATCH * SEQ

NC = 2
NS = 16
NW = NC * NS
TOK_PER_W = NTOK // NW
SEQ_PER_W = TOK_PER_W // SEQ
NJ = EMB // 16
NSLOT = 4
CHUNK_OFFS = tuple(list(range(0, SEQ - 16 + 1, 16)) + [SEQ - 16])
GCHUNKS = ((0, 104), (104, 96))


def _lanesum(x, iota):
    dn = lax.GatherDimensionNumbers(
        offset_dims=(), collapsed_slice_dims=(0,), start_index_map=(0,))
    for k in (8, 4, 2, 1):
        perm = lax.gather(x, (iota ^ k)[:, None], dn, slice_sizes=(1,),
                          mode=lax.GatherScatterMode.PROMISE_IN_BOUNDS)
        x = x + perm
    return x


def _rsqrt(x):
    xi = lax.bitcast_convert_type(x, jnp.int32)
    yi = jnp.int32(0x5F3759DF) - lax.shift_right_arithmetic(xi, 1)
    y = lax.bitcast_convert_type(yi, jnp.float32)
    for _ in range(3):
        y = y * (jnp.float32(1.5) - jnp.float32(0.5) * x * y * y)
    return y


def _body(tok_hbm, sid_hbm, t_hbm, word_hbm, gb_hbm, out_hbm,
          buf_v, idx_vs, sid_vs, ix2_vs, gb_v, sem_i, sem_w, sem_a, sem_o):
    wid = lax.axis_index("s") * NC + lax.axis_index("c")
    base = wid * TOK_PER_W

    pltpu.sync_copy(gb_hbm, gb_v)
    g_regs = tuple(gb_v[0, pl.ds(16 * j, 16)] for j in range(NJ))
    b_regs = tuple(gb_v[1, pl.ds(16 * j, 16)] for j in range(NJ))
    iota16 = lax.iota(jnp.int32, 16)
    iota2 = 2 * iota16

    def ids_copies(s, k):
        tb = base + s * SEQ
        return (pltpu.make_async_copy(tok_hbm.at[pl.ds(tb, SEQ)],
                                      idx_vs[k], sem_i.at[k]),
                pltpu.make_async_copy(sid_hbm.at[pl.ds(tb, SEQ)],
                                      sid_vs[k], sem_i.at[k]))

    def word_copies(k):
        return tuple(
            pltpu.make_async_copy(word_hbm.at[idx_vs[k].at[pl.ds(o, n)]],
                                  buf_v.at[k, pl.ds(o, n)], sem_w.at[k])
            for o, n in GCHUNKS)

    def add_starts(k):
        for o, n in GCHUNKS:
            pltpu.async_copy(t_hbm.at[ix2_vs[k].at[pl.ds(o, n)]],
                             buf_v.at[k, pl.ds(o, n)], sem_a.at[k], add=True)

    def add_waits(k):
        for o, n in GCHUNKS:
            pltpu.make_async_copy(t_hbm.at[ix2_vs[k].at[pl.ds(o, n)]],
                                  buf_v.at[k, pl.ds(o, n)], sem_a.at[k]).wait()

    def out_copy(s, k):
        tb = base + s * SEQ
        return pltpu.make_async_copy(buf_v.at[k],
                                     out_hbm.at[pl.ds(tb, SEQ)], sem_o.at[k])

    def tok_body(i, k):
        g, b = g_regs, b_regs
        x = [buf_v[k, i, pl.ds(16 * j, 16)] for j in range(NJ)]
        s4 = [(x[0] + x[1]) + (x[2] + x[3]), (x[4] + x[5]) + (x[6] + x[7])]
        q4 = [(x[0] * x[0] + x[1] * x[1]) + (x[2] * x[2] + x[3] * x[3]),
              (x[4] * x[4] + x[5] * x[5]) + (x[6] * x[6] + x[7] * x[7])]
        mean = _lanesum(s4[0] + s4[1], iota16) * jnp.float32(1.0 / EMB)
        msq = _lanesum(q4[0] + q4[1], iota16) * jnp.float32(1.0 / EMB)
        var = msq - mean * mean
        istd = _rsqrt(var + jnp.float32(1e-12))
        shift = mean * istd
        for j in range(NJ):
            buf_v[k, i, pl.ds(16 * j, 16)] = (x[j] * istd - shift) * g[j] + b[j]

    def step(v, u):
        s_d, k_d = v + 2, u % NSLOT
        s_a, k_a = v + 1, (u + 3) % NSLOT
        s_b, k_b = v, (u + 2) % NSLOT
        s_c, k_c = v - 1, (u + 1) % NSLOT

        @pl.when(jnp.logical_and(s_d >= 0, s_d < SEQ_PER_W))
        def _():
            for c in ids_copies(s_d, k_d):
                c.start()

        @pl.when(jnp.logical_and(s_a >= 0, s_a < SEQ_PER_W))
        def _():
            for c in ids_copies(s_a, k_a):
                c.wait()
            for off in CHUNK_OFFS:
                ix2_vs[k_a][pl.ds(off, 16)] = (
                    sid_vs[k_a][pl.ds(off, 16)] + (iota2 + jnp.int32(2 * off)))

        @pl.when(jnp.logical_and(s_a >= NSLOT, s_a < SEQ_PER_W))
        def _():
            out_copy(s_a - NSLOT, k_a).wait()

        @pl.when(jnp.logical_and(s_a >= 0, s_a < SEQ_PER_W))
        def _():
            for c in word_copies(k_a):
                c.start()

        @pl.when(jnp.logical_and(s_b >= 0, s_b < SEQ_PER_W))
        def _():
            for c in word_copies(k_b):
                c.wait()
            add_starts(k_b)

        @pl.when(jnp.logical_and(s_c >= 0, s_c < SEQ_PER_W))
        def _():
            add_waits(k_c)
            plsc.parallel_loop(0, SEQ, unroll=4)(
                functools.partial(tok_body, k=k_c))
            out_copy(s_c, k_c).start()

    def quad_body(t, _):
        for u in range(NSLOT):
            step(NSLOT * t + u - 2, u)
        return _

    lax.fori_loop(0, SEQ_PER_W // NSLOT + 1, quad_body, jnp.int32(0))

    for s in range(SEQ_PER_W - NSLOT, SEQ_PER_W):
        out_copy(s, s % NSLOT).wait()


@functools.partial(
    pl.kernel,
    out_type=jax.ShapeDtypeStruct((NTOK, EMB), jnp.float32),
    mesh=plsc.VectorSubcoreMesh(core_axis_name="c", subcore_axis_name="s"),
    scratch_types=[
        pltpu.VMEM((NSLOT, SEQ, EMB), jnp.float32),
        *[pltpu.VMEM((SEQ,), jnp.int32) for _ in range(3 * NSLOT)],
        pltpu.VMEM((2, EMB), jnp.float32),
        pltpu.SemaphoreType.DMA((NSLOT,)),
        pltpu.SemaphoreType.DMA((NSLOT,)),
        pltpu.SemaphoreType.DMA((NSLOT,)),
        pltpu.SemaphoreType.DMA((NSLOT,)),
    ],
    compiler_params=pltpu.CompilerParams(needs_layout_passes=False),
)
def _sc_kernel(tok_hbm, sid_hbm, t_hbm, word_hbm, gb_hbm, out_hbm,
               buf_v, *rest):
    ids = rest[:3 * NSLOT]
    idx_vs, sid_vs, ix2_vs = (ids[0:NSLOT], ids[NSLOT:2 * NSLOT],
                              ids[2 * NSLOT:3 * NSLOT])
    gb_v, sem_i, sem_w, sem_a, sem_o = rest[3 * NSLOT:]
    _body(tok_hbm, sid_hbm, t_hbm, word_hbm, gb_hbm, out_hbm,
          buf_v, idx_vs, sid_vs, ix2_vs, gb_v, sem_i, sem_w, sem_a, sem_o)


def kernel(token_ids, segment_ids, word_emb, position_emb, segment_emb,
           ln_gamma, ln_beta):
    tok = token_ids.reshape(-1).astype(jnp.int32)
    sid = segment_ids.reshape(-1).astype(jnp.int32)
    t_tab = (position_emb[:SEQ, None, :] + segment_emb[None, :, :]).reshape(
        2 * SEQ, EMB)
    gb = jnp.stack([ln_gamma, ln_beta]).astype(jnp.float32)
    out = _sc_kernel(tok, sid, t_tab, word_emb.astype(jnp.float32), gb)
    return out.reshape(BATCH, SEQ, EMB)


# speedup vs baseline: 1.6353x; 1.6353x over previous
import functools

import jax
import jax.numpy as jnp
from jax import lax
from jax.experimental import pallas as pl
from jax.experimental.pallas import tpu as pltpu
from jax.experimental.pallas import tpu_sc as plsc

VOCAB = 100000
EMB = 128
SEQ = 200
BATCH = 4096
NTOK = BATCH * SEQ

NC = 2
NS = 16
NW = NC * NS
TOK_PER_W = NTOK // NW
SEQ_PER_W = TOK_PER_W // SEQ
NJ = EMB // 16
NSLOT = 4
CHUNK_OFFS = tuple(list(range(0, SEQ - 16 + 1, 16)) + [SEQ - 16])
GCHUNKS = ((0, 104), (104, 96))


def _lanesum(x, iota):
    dn = lax.GatherDimensionNumbers(
        offset_dims=(), collapsed_slice_dims=(0,), start_index_map=(0,))
    for k in (8, 4, 2, 1):
        perm = lax.gather(x, (iota ^ k)[:, None], dn, slice_sizes=(1,),
                          mode=lax.GatherScatterMode.PROMISE_IN_BOUNDS)
        x = x + perm
    return x


def _rsqrt(x):
    xi = lax.bitcast_convert_type(x, jnp.int32)
    yi = jnp.int32(0x5F3759DF) - lax.shift_right_arithmetic(xi, 1)
    y = lax.bitcast_convert_type(yi, jnp.float32)
    for _ in range(3):
        y = y * (jnp.float32(1.5) - jnp.float32(0.5) * x * y * y)
    return y


def _body(tok_hbm, sid_hbm, t_hbm, word_hbm, gb_hbm, out_hbm,
          buf_v, idx_vs, sid_vs, ix2_vs, gb_v, t_sh,
          sem_i, sem_w, sem_a, sem_o):
    wid = lax.axis_index("s") * NC + lax.axis_index("c")
    base = wid * TOK_PER_W

    pltpu.sync_copy(gb_hbm, gb_v)

    @pl.when(lax.axis_index("s") == 0)
    def _():
        pltpu.sync_copy(t_hbm, t_sh)
    plsc.subcore_barrier()
    g_regs = tuple(gb_v[0, pl.ds(16 * j, 16)] for j in range(NJ))
    b_regs = tuple(gb_v[1, pl.ds(16 * j, 16)] for j in range(NJ))
    iota16 = lax.iota(jnp.int32, 16)
    iota2 = 2 * iota16

    def ids_copies(s, k):
        tb = base + s * SEQ
        return (pltpu.make_async_copy(tok_hbm.at[pl.ds(tb, SEQ)],
                                      idx_vs[k], sem_i.at[k]),
                pltpu.make_async_copy(sid_hbm.at[pl.ds(tb, SEQ)],
                                      sid_vs[k], sem_i.at[k]))

    def word_copies(k):
        return tuple(
            pltpu.make_async_copy(word_hbm.at[idx_vs[k].at[pl.ds(o, n)]],
                                  buf_v.at[k, pl.ds(o, n)], sem_w.at[k])
            for o, n in GCHUNKS)

    def add_starts(k):
        for o, n in GCHUNKS:
            pltpu.async_copy(t_sh.at[ix2_vs[k].at[pl.ds(o, n)]],
                             buf_v.at[k, pl.ds(o, n)], sem_a.at[k], add=True)

    def add_waits(k):
        for o, n in GCHUNKS:
            pltpu.make_async_copy(t_sh.at[ix2_vs[k].at[pl.ds(o, n)]],
                                  buf_v.at[k, pl.ds(o, n)], sem_a.at[k]).wait()

    def out_copy(s, k):
        tb = base + s * SEQ
        return pltpu.make_async_copy(buf_v.at[k],
                                     out_hbm.at[pl.ds(tb, SEQ)], sem_o.at[k])

    def tok_body(i, k):
        g, b = g_regs, b_regs
        x = [buf_v[k, i, pl.ds(16 * j, 16)] for j in range(NJ)]
        s4 = [(x[0] + x[1]) + (x[2] + x[3]), (x[4] + x[5]) + (x[6] + x[7])]
        q4 = [(x[0] * x[0] + x[1] * x[1]) + (x[2] * x[2] + x[3] * x[3]),
              (x[4] * x[4] + x[5] * x[5]) + (x[6] * x[6] + x[7] * x[7])]
        mean = jnp.sum(s4[0] + s4[1]) * jnp.float32(1.0 / EMB)
        msq = jnp.sum(q4[0] + q4[1]) * jnp.float32(1.0 / EMB)
        var = msq - mean * mean
        istd = _rsqrt(var + jnp.float32(1e-12))
        shift = mean * istd
        for j in range(NJ):
            buf_v[k, i, pl.ds(16 * j, 16)] = (x[j] * istd - shift) * g[j] + b[j]

    def step(v, u):
        s_d, k_d = v + 2, u % NSLOT
        s_a, k_a = v + 1, (u + 3) % NSLOT
        s_b, k_b = v, (u + 2) % NSLOT
        s_c, k_c = v - 1, (u + 1) % NSLOT

        @pl.when(jnp.logical_and(s_d >= 0, s_d < SEQ_PER_W))
        def _():
            for c in ids_copies(s_d, k_d):
                c.start()

        @pl.when(jnp.logical_and(s_a >= 0, s_a < SEQ_PER_W))
        def _():
            for c in ids_copies(s_a, k_a):
                c.wait()
            for off in CHUNK_OFFS:
                ix2_vs[k_a][pl.ds(off, 16)] = (
                    sid_vs[k_a][pl.ds(off, 16)] + (iota2 + jnp.int32(2 * off)))

        @pl.when(jnp.logical_and(s_a >= NSLOT, s_a < SEQ_PER_W))
        def _():
            out_copy(s_a - NSLOT, k_a).wait()

        @pl.when(jnp.logical_and(s_a >= 0, s_a < SEQ_PER_W))
        def _():
            for c in word_copies(k_a):
                c.start()

        @pl.when(jnp.logical_and(s_b >= 0, s_b < SEQ_PER_W))
        def _():
            for c in word_copies(k_b):
                c.wait()
            add_starts(k_b)

        @pl.when(jnp.logical_and(s_c >= 0, s_c < SEQ_PER_W))
        def _():
            add_waits(k_c)
            plsc.parallel_loop(0, SEQ, unroll=4)(
                functools.partial(tok_body, k=k_c))
            out_copy(s_c, k_c).start()

    def quad_body(t, _):
        for u in range(NSLOT):
            step(NSLOT * t + u - 2, u)
        return _

    lax.fori_loop(0, SEQ_PER_W // NSLOT + 1, quad_body, jnp.int32(0))

    for s in range(SEQ_PER_W - NSLOT, SEQ_PER_W):
        out_copy(s, s % NSLOT).wait()


@functools.partial(
    pl.kernel,
    out_type=jax.ShapeDtypeStruct((NTOK, EMB), jnp.float32),
    mesh=plsc.VectorSubcoreMesh(core_axis_name="c", subcore_axis_name="s"),
    scratch_types=[
        pltpu.VMEM((NSLOT, SEQ, EMB), jnp.float32),
        *[pltpu.VMEM((SEQ,), jnp.int32) for _ in range(3 * NSLOT)],
        pltpu.VMEM((2, EMB), jnp.float32),
        pltpu.VMEM_SHARED((2 * SEQ, EMB), jnp.float32),
        pltpu.SemaphoreType.DMA((NSLOT,)),
        pltpu.SemaphoreType.DMA((NSLOT,)),
        pltpu.SemaphoreType.DMA((NSLOT,)),
        pltpu.SemaphoreType.DMA((NSLOT,)),
    ],
    compiler_params=pltpu.CompilerParams(needs_layout_passes=False),
)
def _sc_kernel(tok_hbm, sid_hbm, t_hbm, word_hbm, gb_hbm, out_hbm,
               buf_v, *rest):
    ids = rest[:3 * NSLOT]
    idx_vs, sid_vs, ix2_vs = (ids[0:NSLOT], ids[NSLOT:2 * NSLOT],
                              ids[2 * NSLOT:3 * NSLOT])
    gb_v, t_sh, sem_i, sem_w, sem_a, sem_o = rest[3 * NSLOT:]
    _body(tok_hbm, sid_hbm, t_hbm, word_hbm, gb_hbm, out_hbm,
          buf_v, idx_vs, sid_vs, ix2_vs, gb_v, t_sh,
          sem_i, sem_w, sem_a, sem_o)


def kernel(token_ids, segment_ids, word_emb, position_emb, segment_emb,
           ln_gamma, ln_beta):
    tok = token_ids.reshape(-1).astype(jnp.int32)
    sid = segment_ids.reshape(-1).astype(jnp.int32)
    t_tab = (position_emb[:SEQ, None, :] + segment_emb[None, :, :]).reshape(
        2 * SEQ, EMB)
    gb = jnp.stack([ln_gamma, ln_beta]).astype(jnp.float32)
    out = _sc_kernel(tok, sid, t_tab, word_emb.astype(jnp.float32), gb)
    return out.reshape(BATCH, SEQ, EMB)


# speedup vs baseline: 1.6392x; 1.0024x over previous
import functools

import jax
import jax.numpy as jnp
from jax import lax
from jax.experimental import pallas as pl
from jax.experimental.pallas import tpu as pltpu
from jax.experimental.pallas import tpu_sc as plsc

VOCAB = 100000
EMB = 128
SEQ = 200
BATCH = 4096
NTOK = BATCH * SEQ

NC = 2
NS = 16
NW = NC * NS
TOK_PER_W = NTOK // NW
SEQ_PER_W = TOK_PER_W // SEQ
NJ = EMB // 16
NSLOT = 4
CHUNK_OFFS = tuple(list(range(0, SEQ - 16 + 1, 16)) + [SEQ - 16])
GCHUNKS = ((0, 104), (104, 96))


def _lanesum(x, iota):
    dn = lax.GatherDimensionNumbers(
        offset_dims=(), collapsed_slice_dims=(0,), start_index_map=(0,))
    for k in (8, 4, 2, 1):
        perm = lax.gather(x, (iota ^ k)[:, None], dn, slice_sizes=(1,),
                          mode=lax.GatherScatterMode.PROMISE_IN_BOUNDS)
        x = x + perm
    return x


def _rsqrt(x):
    xi = lax.bitcast_convert_type(x, jnp.int32)
    yi = jnp.int32(0x5F3759DF) - lax.shift_right_arithmetic(xi, 1)
    y = lax.bitcast_convert_type(yi, jnp.float32)
    for _ in range(3):
        y = y * (jnp.float32(1.5) - jnp.float32(0.5) * x * y * y)
    return y


def _body(tok_hbm, sid_hbm, t_hbm, word_hbm, gb_hbm, out_hbm,
          buf_v, idx_vs, sid_vs, ix2_vs, gb_v, t_sh,
          sem_i, sem_w, sem_a, sem_o):
    wid = lax.axis_index("s") * NC + lax.axis_index("c")
    base = wid * TOK_PER_W

    pltpu.sync_copy(gb_hbm, gb_v)

    @pl.when(lax.axis_index("s") == 0)
    def _():
        pltpu.sync_copy(t_hbm, t_sh)
    plsc.subcore_barrier()
    g_regs = tuple(gb_v[0, pl.ds(16 * j, 16)] for j in range(NJ))
    b_regs = tuple(gb_v[1, pl.ds(16 * j, 16)] for j in range(NJ))
    iota16 = lax.iota(jnp.int32, 16)
    iota2 = 2 * iota16

    def ids_copies(s, k):
        tb = base + s * SEQ
        return (pltpu.make_async_copy(tok_hbm.at[pl.ds(tb, SEQ)],
                                      idx_vs[k], sem_i.at[k]),
                pltpu.make_async_copy(sid_hbm.at[pl.ds(tb, SEQ)],
                                      sid_vs[k], sem_i.at[k]))

    def word_copies(k):
        return tuple(
            pltpu.make_async_copy(word_hbm.at[idx_vs[k].at[pl.ds(o, n)]],
                                  buf_v.at[k, pl.ds(o, n)], sem_w.at[k])
            for o, n in GCHUNKS)

    def add_starts(k):
        for o, n in GCHUNKS:
            pltpu.async_copy(t_sh.at[ix2_vs[k].at[pl.ds(o, n)]],
                             buf_v.at[k, pl.ds(o, n)], sem_a.at[k], add=True)

    def add_waits(k):
        for o, n in GCHUNKS:
            pltpu.make_async_copy(t_sh.at[ix2_vs[k].at[pl.ds(o, n)]],
                                  buf_v.at[k, pl.ds(o, n)], sem_a.at[k]).wait()

    def out_copy(s, k):
        tb = base + s * SEQ
        return pltpu.make_async_copy(buf_v.at[k],
                                     out_hbm.at[pl.ds(tb, SEQ)], sem_o.at[k])

    def tok_body(i, k):
        g, b = g_regs, b_regs
        x = [buf_v[k, i, pl.ds(16 * j, 16)] for j in range(NJ)]
        s4 = [(x[0] + x[1]) + (x[2] + x[3]), (x[4] + x[5]) + (x[6] + x[7])]
        q4 = [(x[0] * x[0] + x[1] * x[1]) + (x[2] * x[2] + x[3] * x[3]),
              (x[4] * x[4] + x[5] * x[5]) + (x[6] * x[6] + x[7] * x[7])]
        mean = jnp.sum(s4[0] + s4[1]) * jnp.float32(1.0 / EMB)
        msq = jnp.sum(q4[0] + q4[1]) * jnp.float32(1.0 / EMB)
        var = msq - mean * mean
        istd = _rsqrt(var + jnp.float32(1e-12))
        shift = mean * istd
        for j in range(NJ):
            buf_v[k, i, pl.ds(16 * j, 16)] = (x[j] * istd - shift) * g[j] + b[j]

    def step(v, u):
        s_d, k_d = v + 3, (u + 3) % NSLOT
        s_a, k_a = v + 2, (u + 2) % NSLOT
        s_b, k_b = v + 1, (u + 1) % NSLOT
        s_c, k_c = v, u % NSLOT

        @pl.when(jnp.logical_and(s_d >= 0, s_d < SEQ_PER_W))
        def _():
            for c in ids_copies(s_d, k_d):
                c.start()

        @pl.when(jnp.logical_and(s_a >= 0, s_a < SEQ_PER_W))
        def _():
            for c in ids_copies(s_a, k_a):
                c.wait()
            for off in CHUNK_OFFS:
                ix2_vs[k_a][pl.ds(off, 16)] = (
                    sid_vs[k_a][pl.ds(off, 16)] + (iota2 + jnp.int32(2 * off)))

        @pl.when(jnp.logical_and(s_a >= NSLOT, s_a < SEQ_PER_W))
        def _():
            out_copy(s_a - NSLOT, k_a).wait()

        @pl.when(jnp.logical_and(s_a >= 0, s_a < SEQ_PER_W))
        def _():
            for c in word_copies(k_a):
                c.start()

        @pl.when(jnp.logical_and(s_b >= 0, s_b < SEQ_PER_W))
        def _():
            for c in word_copies(k_b):
                c.wait()
            add_starts(k_b)

        @pl.when(jnp.logical_and(s_c >= 0, s_c < SEQ_PER_W))
        def _():
            add_waits(k_c)
            plsc.parallel_loop(0, SEQ, unroll=4)(
                functools.partial(tok_body, k=k_c))
            out_copy(s_c, k_c).start()

    def quad_body(t, _):
        for u in range(NSLOT):
            step(NSLOT * t + u - NSLOT, u)
        return _

    lax.fori_loop(0, SEQ_PER_W // NSLOT + 1, quad_body, jnp.int32(0))

    for s in range(SEQ_PER_W - NSLOT, SEQ_PER_W):
        out_copy(s, s % NSLOT).wait()


@functools.partial(
    pl.kernel,
    out_type=jax.ShapeDtypeStruct((NTOK, EMB), jnp.float32),
    mesh=plsc.VectorSubcoreMesh(core_axis_name="c", subcore_axis_name="s"),
    scratch_types=[
        pltpu.VMEM((NSLOT, SEQ, EMB), jnp.float32),
        *[pltpu.VMEM((SEQ,), jnp.int32) for _ in range(3 * NSLOT)],
        pltpu.VMEM((2, EMB), jnp.float32),
        pltpu.VMEM_SHARED((2 * SEQ, EMB), jnp.float32),
        pltpu.SemaphoreType.DMA((NSLOT,)),
        pltpu.SemaphoreType.DMA((NSLOT,)),
        pltpu.SemaphoreType.DMA((NSLOT,)),
        pltpu.SemaphoreType.DMA((NSLOT,)),
    ],
    compiler_params=pltpu.CompilerParams(needs_layout_passes=False),
)
def _sc_kernel(tok_hbm, sid_hbm, t_hbm, word_hbm, gb_hbm, out_hbm,
               buf_v, *rest):
    ids = rest[:3 * NSLOT]
    idx_vs, sid_vs, ix2_vs = (ids[0:NSLOT], ids[NSLOT:2 * NSLOT],
                              ids[2 * NSLOT:3 * NSLOT])
    gb_v, t_sh, sem_i, sem_w, sem_a, sem_o = rest[3 * NSLOT:]
    _body(tok_hbm, sid_hbm, t_hbm, word_hbm, gb_hbm, out_hbm,
          buf_v, idx_vs, sid_vs, ix2_vs, gb_v, t_sh,
          sem_i, sem_w, sem_a, sem_o)


def kernel(token_ids, segment_ids, word_emb, position_emb, segment_emb,
           ln_gamma, ln_beta):
    tok = token_ids.reshape(-1).astype(jnp.int32)
    sid = segment_ids.reshape(-1).astype(jnp.int32)
    t_tab = (position_emb[:SEQ, None, :] + segment_emb[None, :, :]).reshape(
        2 * SEQ, EMB)
    gb = jnp.stack([ln_gamma, ln_beta]).astype(jnp.float32)
    out = _sc_kernel(tok, sid, t_tab, word_emb.astype(jnp.float32), gb)
    return out.reshape(BATCH, SEQ, EMB)
